# baseline (speedup 1.0000x reference)
import functools

import jax
import jax.numpy as jnp
from jax import lax
from jax.experimental import pallas as pl
from jax.experimental.pallas import tpu as pltpu
from jax.experimental.pallas import tpu_sc as plsc

N = 10000
NPAD = 10240
H = 128
C = 64
EPS = 0.1
NC = 2
NS = 16
NT = NC * NS
CHUNK = 96
LANES = 16
NBUF = 2
RPT = NPAD // NS
ZROWS = 80


def _tc1_body(x_ref, w0_ref, b0_ref, g_ref, h_ref, x12_ref):
    h = jnp.dot(x_ref[...], w0_ref[...], preferred_element_type=jnp.float32)
    h = jnp.maximum(h + b0_ref[...], 0.0)
    h_ref[...] = h
    x12_ref[...] = lax.dot_general(
        g_ref[...], h, (((1,), (1,)), ((), ())),
        preferred_element_type=jnp.float32)


def _tc2_body(r_ref, h0_ref, g_ref, h_ref, x12_ref):
    hn = EPS * h0_ref[...] + r_ref[0, :N] + r_ref[1, :N]
    h_ref[...] = hn
    x12_ref[...] = lax.dot_general(
        g_ref[...], hn, (((1,), (1,)), ((), ())),
        preferred_element_type=jnp.float32)


def _tc3_body(r_ref, h0_ref, w1_ref, b1_ref, o_ref):
    hn = EPS * h0_ref[...] + r_ref[0, :N] + r_ref[1, :N]
    o_ref[...] = jnp.dot(hn, w1_ref[...],
                         preferred_element_type=jnp.float32) + b1_ref[...]


_tc1 = pl.pallas_call(
    _tc1_body,
    out_shape=[jax.ShapeDtypeStruct((N, H), jnp.float32),
               jax.ShapeDtypeStruct((8, N), jnp.float32)],
)

_tc2 = pl.pallas_call(
    _tc2_body,
    out_shape=[jax.ShapeDtypeStruct((N, H), jnp.float32),
               jax.ShapeDtypeStruct((8, N), jnp.float32)],
)

_tc3 = pl.pallas_call(
    _tc3_body,
    out_shape=jax.ShapeDtypeStruct((N, C), jnp.float32),
)


NFILL = 4


def _sc_edge_body(nchunk, src_h, dst_h, adj_h, x12_h, h_h, out_h,
                  x1_v, x2_v, s0, s1, s2, s3, d0, d1, d2, d3,
                  a0, a1, a2, a3, m_v,
                  r0, r1, res_sh, sf0, sf1, sf2, sf3, sg0, sg1):
    sidx = (s0, s1, s2, s3)
    didx = (d0, d1, d2, d3)
    adjb = (a0, a1, a2, a3)
    rows = (r0, r1)
    sf = (sf0, sf1, sf2, sf3)
    sg = (sg0, sg1)
    c = lax.axis_index("c")
    s = lax.axis_index("s")
    wid = c * NS + s
    ept = nchunk * CHUNK

    pltpu.sync_copy(x12_h.at[0], x1_v)
    pltpu.sync_copy(x12_h.at[1], x2_v)

    zero16 = jnp.zeros((LANES,), jnp.float32)

    def _zrow(i, carry):
        for g in range(H // LANES):
            r0[i, pl.ds(g * LANES, LANES)] = zero16
        return carry

    lax.fori_loop(0, ZROWS, _zrow, 0)
    for k in range(RPT // ZROWS):
        pltpu.sync_copy(r0.at[pl.ds(0, ZROWS)],
                        res_sh.at[pl.ds(s * RPT + k * ZROWS, ZROWS)])

    def _fill_start(ci, f):
        base = wid * ept + ci * CHUNK
        pltpu.async_copy(src_h.at[pl.ds(base, CHUNK)], sidx[f], sf[f])
        pltpu.async_copy(dst_h.at[pl.ds(base, CHUNK)], didx[f], sf[f])
        pltpu.async_copy(adj_h.at[pl.ds(base, CHUNK)], adjb[f], sf[f])

    def _fill_wait(f):
        pltpu.make_async_copy(src_h.at[pl.ds(0, CHUNK)], sidx[f], sf[f]).wait()
        pltpu.make_async_copy(dst_h.at[pl.ds(0, CHUNK)], didx[f], sf[f]).wait()
        pltpu.make_async_copy(adj_h.at[pl.ds(0, CHUNK)], adjb[f], sf[f]).wait()

    for f in range(NBUF + 1):
        _fill_start(f, f)
    for b in range(NBUF):
        _fill_wait(b)
        pltpu.async_copy(h_h.at[didx[b]], rows[b], sg[b])
    plsc.subcore_barrier()

    def _outer(t, carry):
        for j in range(NFILL):
            ci = t * NFILL + j
            b = j % NBUF
            f = j
            fp = (j + NBUF + 1) % NFILL
            rb = rows[b]
            @pl.when(ci + NBUF + 1 < nchunk)
            def _fill_ahead():
                _fill_start(ci + NBUF + 1, fp)
            pltpu.make_async_copy(h_h.at[didx[b]], rb, sg[b]).wait()
            for g in range(CHUNK // LANES):
                sl = pl.ds(g * LANES, LANES)
                sv = sidx[f][sl]
                dv = didx[f][sl]
                av = adjb[f][sl]
                z = plsc.load_gather(x1_v, [sv]) + plsc.load_gather(x2_v, [dv])
                az = jnp.abs(z)
                e = jnp.exp(az * (-2.0))
                m_v[sl] = jnp.sign(z) * ((1.0 - e) / (1.0 + e)) * av
            def _scale(ei, carry2):
                mb = plsc.load_gather(m_v, [jnp.zeros((LANES,), jnp.int32) + ei])
                for g in range(H // LANES):
                    sl = pl.ds(g * LANES, LANES)
                    rb[ei, sl] = rb[ei, sl] * mb
                return carry2

            lax.fori_loop(0, CHUNK, _scale, 0)
            pltpu.sync_copy(rb, res_sh.at[sidx[f]], add=True)
            nxt = ci + NBUF
            fn = (j + NBUF) % NFILL
            @pl.when(nxt < nchunk)
            def _prefetch():
                _fill_wait(fn)
                pltpu.async_copy(h_h.at[didx[fn]], rb, sg[b])
        return carry

    lax.fori_loop(0, nchunk // NFILL, _outer, 0)
    plsc.subcore_barrier()

    for k in range(RPT // ZROWS):
        r0c = s * RPT + k * ZROWS
        pltpu.sync_copy(res_sh.at[pl.ds(r0c, ZROWS)],
                        out_h.at[c, pl.ds(r0c, ZROWS)])


@functools.cache
def _make_sc_kernel(nchunk):
    mesh = plsc.VectorSubcoreMesh(core_axis_name="c", subcore_axis_name="s",
                                  num_cores=NC, num_subcores=NS)
    return pl.kernel(
        functools.partial(_sc_edge_body, nchunk),
        out_type=jax.ShapeDtypeStruct((NC, NPAD, H), jnp.float32),
        mesh=mesh,
        compiler_params=pltpu.CompilerParams(needs_layout_passes=False),
        scratch_types=[
            pltpu.VMEM((N,), jnp.float32),
            pltpu.VMEM((N,), jnp.float32),
            pltpu.VMEM((CHUNK,), jnp.int32),
            pltpu.VMEM((CHUNK,), jnp.int32),
            pltpu.VMEM((CHUNK,), jnp.int32),
            pltpu.VMEM((CHUNK,), jnp.int32),
            pltpu.VMEM((CHUNK,), jnp.int32),
            pltpu.VMEM((CHUNK,), jnp.int32),
            pltpu.VMEM((CHUNK,), jnp.int32),
            pltpu.VMEM((CHUNK,), jnp.int32),
            pltpu.VMEM((CHUNK,), jnp.float32),
            pltpu.VMEM((CHUNK,), jnp.float32),
            pltpu.VMEM((CHUNK,), jnp.float32),
            pltpu.VMEM((CHUNK,), jnp.float32),
            pltpu.VMEM((CHUNK,), jnp.float32),
            pltpu.VMEM((CHUNK, H), jnp.float32),
            pltpu.VMEM((CHUNK, H), jnp.float32),
            pltpu.VMEM_SHARED((NPAD, H), jnp.float32),
            pltpu.SemaphoreType.DMA,
            pltpu.SemaphoreType.DMA,
            pltpu.SemaphoreType.DMA,
            pltpu.SemaphoreType.DMA,
            pltpu.SemaphoreType.DMA,
            pltpu.SemaphoreType.DMA,
        ],
    )


def kernel(x, edge_index, adj_vals, W0, b0, W1, b1, g1, g2):
    src = edge_index[0].astype(jnp.int32)
    dst = edge_index[1].astype(jnp.int32)
    e_total = src.shape[0]
    nchunk = -(-e_total // (NT * CHUNK))
    nchunk = -(-nchunk // 4) * 4
    epad = nchunk * CHUNK * NT
    pad = epad - e_total
    if pad:
        src = jnp.concatenate([src, jnp.zeros((pad,), jnp.int32)])
        dst = jnp.concatenate([dst, jnp.zeros((pad,), jnp.int32)])
        adj = jnp.concatenate([adj_vals, jnp.zeros((pad,), jnp.float32)])
    else:
        adj = adj_vals

    zpad = jnp.zeros((6, H), jnp.float32)
    g_a = jnp.concatenate([g1[0:1], g2[0:1], zpad])
    g_b = jnp.concatenate([g1[1:2], g2[1:2], zpad])

    sc_k = _make_sc_kernel(nchunk)

    h0, x12 = _tc1(x, W0, b0[None, :], g_a)
    res = sc_k(src, dst, adj, x12, h0)
    h1, x12 = _tc2(res, h0, g_b)
    res = sc_k(src, dst, adj, x12, h1)
    return _tc3(res, h0, W1, b1[None, :])


# speedup vs baseline: 3.2103x; 3.2103x over previous
import functools

import jax
import jax.numpy as jnp
from jax import lax
from jax.experimental import pallas as pl
from jax.experimental.pallas import tpu as pltpu
from jax.experimental.pallas import tpu_sc as plsc

N = 10000
NPAD = 10240
H = 128
C = 64
EPS = 0.1
NC = 2
NS = 16
NT = NC * NS
CHUNK = 48
LANES = 16
NRING = 4
RPT = NPAD // NS
ZROWS = 40


def _tc1_body(x_ref, w0_ref, b0_ref, g_ref, h_ref, x12_ref):
    h = jnp.dot(x_ref[...], w0_ref[...], preferred_element_type=jnp.float32)
    h = jnp.maximum(h + b0_ref[...], 0.0)
    h_ref[...] = h
    x12_ref[...] = lax.dot_general(
        g_ref[...], h, (((1,), (1,)), ((), ())),
        preferred_element_type=jnp.float32)


def _tc2_body(r_ref, h0_ref, g_ref, h_ref, x12_ref):
    hn = EPS * h0_ref[...] + r_ref[0, :N] + r_ref[1, :N]
    h_ref[...] = hn
    x12_ref[...] = lax.dot_general(
        g_ref[...], hn, (((1,), (1,)), ((), ())),
        preferred_element_type=jnp.float32)


def _tc3_body(r_ref, h0_ref, w1_ref, b1_ref, o_ref):
    hn = EPS * h0_ref[...] + r_ref[0, :N] + r_ref[1, :N]
    o_ref[...] = jnp.dot(hn, w1_ref[...],
                         preferred_element_type=jnp.float32) + b1_ref[...]


_tc1 = pl.pallas_call(
    _tc1_body,
    out_shape=[jax.ShapeDtypeStruct((N, H), jnp.float32),
               jax.ShapeDtypeStruct((8, N), jnp.float32)],
)

_tc2 = pl.pallas_call(
    _tc2_body,
    out_shape=[jax.ShapeDtypeStruct((N, H), jnp.float32),
               jax.ShapeDtypeStruct((8, N), jnp.float32)],
)

_tc3 = pl.pallas_call(
    _tc3_body,
    out_shape=jax.ShapeDtypeStruct((N, C), jnp.float32),
)


def _sc_edge_body(nc0, nc1, src_h, dst_h, adj_h, x12_h, h_h, out_h,
                  x1_v, x2_v, sidx, didx, adjb, m_v, rows,
                  res_sh, sf, sg, ss):
    c = lax.axis_index("c")
    s = lax.axis_index("s")
    ncb = jnp.where(c == 0, nc0, nc1)
    cbase = jnp.where(c == 0, s * nc0, NS * nc0 + s * nc1)

    pltpu.sync_copy(x12_h.at[0], x1_v)
    pltpu.sync_copy(x12_h.at[1], x2_v)

    zero16 = jnp.zeros((LANES,), jnp.float32)

    def _zrow(i, carry):
        for g in range(H // LANES):
            rows[0][i, pl.ds(g * LANES, LANES)] = zero16
        return carry

    lax.fori_loop(0, ZROWS, _zrow, 0)
    for k in range(RPT // ZROWS):
        pltpu.sync_copy(rows[0].at[pl.ds(0, ZROWS)],
                        res_sh.at[pl.ds(s * RPT + k * ZROWS, ZROWS)])

    def _fill_start(ci, f):
        base = (cbase + ci) * CHUNK
        pltpu.async_copy(src_h.at[pl.ds(base, CHUNK)], sidx[f], sf[f])
        pltpu.async_copy(dst_h.at[pl.ds(base, CHUNK)], didx[f], sf[f])
        pltpu.async_copy(adj_h.at[pl.ds(base, CHUNK)], adjb[f], sf[f])

    def _fill_wait(f):
        pltpu.make_async_copy(src_h.at[pl.ds(0, CHUNK)], sidx[f], sf[f]).wait()
        pltpu.make_async_copy(dst_h.at[pl.ds(0, CHUNK)], didx[f], sf[f]).wait()
        pltpu.make_async_copy(adj_h.at[pl.ds(0, CHUNK)], adjb[f], sf[f]).wait()

    for f in range(3):
        _fill_start(f, f)
    for b in range(2):
        _fill_wait(b)
        pltpu.async_copy(h_h.at[didx[b]], rows[b], sg[b])
    plsc.subcore_barrier()

    def _outer(t, carry):
        for b in range(NRING):
            ci = t * NRING + b
            b3 = (b + 3) % NRING
            b2 = (b + 2) % NRING
            rb = rows[b]
            pltpu.make_async_copy(h_h.at[didx[b]], rb, sg[b]).wait()
            for g in range(CHUNK // LANES):
                sl = pl.ds(g * LANES, LANES)
                sv = sidx[b][sl]
                dv = didx[b][sl]
                av = adjb[b][sl]
                z = plsc.load_gather(x1_v, [sv]) + plsc.load_gather(x2_v, [dv])
                az = jnp.abs(z)
                e = jnp.exp(az * (-2.0))
                m_v[sl] = jnp.sign(z) * ((1.0 - e) / (1.0 + e)) * av
            @plsc.parallel_loop(0, CHUNK, step=1, unroll=4)
            def _scale(ei):
                mb = plsc.load_gather(m_v, [jnp.zeros((LANES,), jnp.int32) + ei])
                for g in range(H // LANES):
                    sl = pl.ds(g * LANES, LANES)
                    rb[ei, sl] = rb[ei, sl] * mb
            pltpu.async_copy(rb, res_sh.at[sidx[b]], ss[b], add=True)
            @pl.when(ci >= 1)
            def _drain():
                pltpu.make_async_copy(rows[b3], res_sh.at[sidx[b3]],
                                      ss[b3]).wait()
            @pl.when(ci + 3 < ncb)
            def _fill_ahead():
                _fill_start(ci + 3, b3)
            @pl.when(ci + 2 < ncb)
            def _prefetch():
                _fill_wait(b2)
                pltpu.async_copy(h_h.at[didx[b2]], rows[b2], sg[b2])
        return carry

    lax.fori_loop(0, ncb // NRING, _outer, 0)
    pltpu.make_async_copy(rows[NRING - 1], res_sh.at[sidx[NRING - 1]],
                          ss[NRING - 1]).wait()
    plsc.subcore_barrier()

    for k in range(RPT // ZROWS):
        r0c = s * RPT + k * ZROWS
        pltpu.sync_copy(res_sh.at[pl.ds(r0c, ZROWS)],
                        out_h.at[c, pl.ds(r0c, ZROWS)])


@functools.cache
def _make_sc_kernel(nc0, nc1):
    mesh = plsc.VectorSubcoreMesh(core_axis_name="c", subcore_axis_name="s",
                                  num_cores=NC, num_subcores=NS)
    return pl.kernel(
        functools.partial(_sc_edge_body, nc0, nc1),
        out_type=jax.ShapeDtypeStruct((NC, NPAD, H), jnp.float32),
        mesh=mesh,
        compiler_params=pltpu.CompilerParams(needs_layout_passes=False),
        scratch_types=[
            pltpu.VMEM((N,), jnp.float32),
            pltpu.VMEM((N,), jnp.float32),
            [pltpu.VMEM((CHUNK,), jnp.int32)] * NRING,
            [pltpu.VMEM((CHUNK,), jnp.int32)] * NRING,
            [pltpu.VMEM((CHUNK,), jnp.float32)] * NRING,
            pltpu.VMEM((CHUNK,), jnp.float32),
            [pltpu.VMEM((CHUNK, H), jnp.float32)] * NRING,
            pltpu.VMEM_SHARED((NPAD, H), jnp.float32),
            [pltpu.SemaphoreType.DMA] * NRING,
            [pltpu.SemaphoreType.DMA] * NRING,
            [pltpu.SemaphoreType.DMA] * NRING,
        ],
    )


def kernel(x, edge_index, adj_vals, W0, b0, W1, b1, g1, g2):
    src = edge_index[0].astype(jnp.int32)
    dst = edge_index[1].astype(jnp.int32)
    e_total = src.shape[0]
    total = -(-e_total // (NS * CHUNK))
    nc1 = max(NRING, (total * 37 // 100) // NRING * NRING)
    nc0 = -(-(total - nc1) // NRING) * NRING
    epad = NS * (nc0 + nc1) * CHUNK
    pad = epad - e_total
    if pad:
        src = jnp.concatenate([src, jnp.zeros((pad,), jnp.int32)])
        dst = jnp.concatenate([dst, jnp.zeros((pad,), jnp.int32)])
        adj = jnp.concatenate([adj_vals, jnp.zeros((pad,), jnp.float32)])
    else:
        adj = adj_vals

    zpad = jnp.zeros((6, H), jnp.float32)
    g_a = jnp.concatenate([g1[0:1], g2[0:1], zpad])
    g_b = jnp.concatenate([g1[1:2], g2[1:2], zpad])

    sc_k = _make_sc_kernel(nc0, nc1)

    h0, x12 = _tc1(x, W0, b0[None, :], g_a)
    res = sc_k(src, dst, adj, x12, h0)
    h1, x12 = _tc2(res, h0, g_b)
    res = sc_k(src, dst, adj, x12, h1)
    return _tc3(res, h0, W1, b1[None, :])
